# baseline (speedup 1.0000x reference)
import functools

import numpy as np
import jax
import jax.numpy as jnp
from jax import lax
from jax.experimental import pallas as pl
from jax.experimental.pallas import tpu as pltpu
from jax.experimental.pallas import tpu_sc as plsc

F32 = jnp.float32
I32 = jnp.int32

NT = 16
NQ = 512
NR = NT * NQ
NK = 50000
E = 131072
DKEY = 128
DOUT = 112
TDIM = 128
NB = 16
NPRE = 12
NTILE = 16
CH = 128
NCH = E // (NTILE * CH)
ACC_W = 128
DCH = 64
DNCH = E // (NTILE * DCH)


def _sigmoid(x):
    return 1.0 / (1.0 + jnp.exp(-x))


def _stage_a(Ts, time2, qxT, vcomp, Wqt, bqt, Wdst, bdst, freqs,
             qn_ref, qpos_ref, vrot_ref, tdst_ref):
    q = Ts[:, 0:4]
    nrm = jnp.sqrt(jnp.sum(q * q, axis=1, keepdims=True))
    qn = q / (nrm + 1e-8)
    qn_ref[...] = qn
    qw, qx, qy, qz = qn[:, 0:1], qn[:, 1:2], qn[:, 2:3], qn[:, 3:4]

    arg = time2[...] * freqs[...]
    te = jnp.concatenate([jnp.sin(arg), jnp.cos(arg)], axis=1)
    qt = jnp.dot(te, Wqt[...], preferred_element_type=F32) + bqt[...]
    tdst_ref[...] = jnp.dot(qt, Wdst[...], preferred_element_type=F32) + bdst[...]

    def rot(vx, vy, vz):
        tx = 2.0 * (qy * vz - qz * vy)
        ty = 2.0 * (qz * vx - qx * vz)
        tz = 2.0 * (qx * vy - qy * vx)
        ox = vx + qw * tx + (qy * tz - qz * ty)
        oy = vy + qw * ty + (qz * tx - qx * tz)
        oz = vz + qw * tz + (qx * ty - qy * tx)
        return ox, oy, oz

    px, py, pz = rot(qxT[0:1, :], qxT[1:2, :], qxT[2:3, :])
    qpos_ref[0] = px + Ts[:, 4:5]
    qpos_ref[1] = py + Ts[:, 5:6]
    qpos_ref[2] = pz + Ts[:, 6:7]

    for k in range(8):
        ox, oy, oz = rot(vcomp[0, k:k + 1, :], vcomp[1, k:k + 1, :],
                         vcomp[2, k:k + 1, :])
        vrot_ref[0, k] = ox
        vrot_ref[1, k] = oy
        vrot_ref[2, k] = oz


def _stage_ak(kf0, kf1, Wmsg, out0_ref, out1_ref):
    z = jnp.zeros((kf0.shape[0], ACC_W - DOUT), F32)
    p0 = jnp.dot(kf0[...], Wmsg[0], preferred_element_type=F32)
    out0_ref[...] = jnp.concatenate([p0, z], axis=1)
    p1 = jnp.dot(kf1[...], Wmsg[1], preferred_element_type=F32)
    out1_ref[...] = jnp.concatenate([p1, z], axis=1)


def _stage_b(kxc_h, kyc_h, kzc_h, qx0_h, qx1_h, qx2_h, esrc_h, edst_h, out_h,
             src_v, dst_v, kc0, kc1, qc0, qc1, d2_v,
             kx_sh, ky_sh, kz_sh, qx_sh, qy_sh, qz_sh,
             sem0, sem1):
    c = lax.axis_index("c")
    s = lax.axis_index("s")
    pltpu.sync_copy(esrc_h.at[c, s], src_v)
    pltpu.sync_copy(edst_h.at[c, s], dst_v)

    @pl.when(s == 0)
    def _():
        pltpu.sync_copy(kxc_h, kx_sh)
        pltpu.sync_copy(kyc_h, ky_sh)
        pltpu.sync_copy(kzc_h, kz_sh)
        pltpu.sync_copy(qx0_h, qx_sh)
        pltpu.sync_copy(qx1_h, qy_sh)
        pltpu.sync_copy(qx2_h, qz_sh)

    plsc.subcore_barrier()
    sems = (sem0, sem1)
    kcs = (kc0, kc1)
    qcs = (qc0, qc1)
    ktabs = (kx_sh, ky_sh, kz_sh)
    qtabs = (qx_sh, qy_sh, qz_sh)

    def streams(ch, b):
        out = []
        for j in range(3):
            out.append(pltpu.make_async_copy(
                ktabs[j].at[src_v.at[ch]],
                kcs[b].at[pl.ds(j * CH, CH)], sems[b]))
            out.append(pltpu.make_async_copy(
                qtabs[j].at[dst_v.at[ch]],
                qcs[b].at[pl.ds(j * CH, CH)], sems[b]))
        return out

    for b in range(2):
        for st in streams(b, b):
            st.start()

    def body(g, carry):
        for b in range(2):
            ch = 2 * g + b
            for st in streams(ch, b):
                st.wait()
            for grp in range(8):
                o = grp * 16
                kv = kcs[b]
                qv = qcs[b]
                ddx = kv[pl.ds(o, 16)] - qv[pl.ds(o, 16)]
                ddy = kv[pl.ds(CH + o, 16)] - qv[pl.ds(CH + o, 16)]
                ddz = kv[pl.ds(2 * CH + o, 16)] - qv[pl.ds(2 * CH + o, 16)]
                d2_v[ch, pl.ds(o, 16)] = ddx * ddx + ddy * ddy + ddz * ddz
            nxt = ch + 2

            @pl.when(nxt < NCH)
            def _():
                for st in streams(nxt, b):
                    st.start()
        return carry

    lax.fori_loop(0, NCH // 2, body, 0)
    pltpu.sync_copy(d2_v, out_h.at[c, s])


def _stage_c(d2, Wr, cbs, isg, out_ref):
    d = jnp.sqrt(d2[0, 0])
    ds = d * isg[0]
    diff = ds - cbs[0]
    basisT = jnp.exp(-0.5 * diff * diff)
    out_ref[0] = lax.dot_general(
        basisT, Wr[0], dimension_numbers=(((0,), (0,)), ((), ())),
        preferred_element_type=F32)


def _stage_d(kp0_h, kp1_h, gate_h, esrc_h, edst_h, out_h,
             srcr, dstr, rows_v, gate_v, msg_v, acc_sh,
             sr0, sr1, sg, si0, si1, sd0, sd1, ss0, ss1):
    c = lax.axis_index("c")
    s = lax.axis_index("s")
    z16 = jnp.zeros((16,), F32)

    def zbody(r, carry):
        for j in range(8):
            msg_v[0, r, pl.ds(j * 16, 16)] = z16
        return carry

    lax.fori_loop(0, DCH, zbody, 0)
    for i in range(8):
        pltpu.sync_copy(msg_v.at[0], acc_sh.at[pl.ds(s * 512 + i * DCH, DCH)])
    plsc.subcore_barrier()
    ones_col = jnp.where(lax.iota(I32, 16) == 0, 1.0, 0.0).astype(F32)

    srs = (sr0, sr1)
    sis = (si0, si1)
    sds = (sd0, sd1)
    sss = (ss0, ss1)
    gtab = gate_h.at[c]

    def idx_copies(ch, b):
        q = ch % 4
        return (pltpu.make_async_copy(esrc_h.at[c, s, ch], srcr.at[q],
                                      sis[b]),
                pltpu.make_async_copy(edst_h.at[c, s, ch], dstr.at[q],
                                     sds[b]))

    def fire_rows(ch, b):
        idx = srcr.at[ch % 4, 0]

        @pl.when(c == 0)
        def _():
            pltpu.async_copy(kp0_h.at[idx], rows_v.at[b], srs[b])

        @pl.when(c == 1)
        def _():
            pltpu.async_copy(kp1_h.at[idx], rows_v.at[b], srs[b])

    def fire_gate(ch):
        pltpu.async_copy(gtab.at[pl.ds(s * DNCH * DCH + ch * DCH, DCH)],
                         gate_v, sg)

    for ch0 in range(2):
        for cp in idx_copies(ch0, ch0):
            cp.start()
    for ch0 in range(2):
        for cp in idx_copies(ch0, ch0):
            cp.wait()
        fire_rows(ch0, ch0)
    fire_gate(0)

    def body(g, carry):
        for b in range(2):
            ch = 2 * g + b
            pltpu.make_async_copy(kp0_h.at[srcr.at[ch % 4, 0]], rows_v.at[b],
                                  srs[b]).wait()
            pltpu.make_async_copy(
                gtab.at[pl.ds(s * DNCH * DCH + ch * DCH, DCH)],
                gate_v, sg).wait()

            @pl.when(ch >= 2)
            def _():
                pltpu.make_async_copy(
                    msg_v.at[b], acc_sh.at[dstr.at[ch % 4, 0]], sss[b]).wait()

            @pl.when(ch + 2 < DNCH)
            def _():
                for cp in idx_copies(ch + 2, b):
                    cp.start()

            def rbody(r, rcarry):
                for j in range(7):
                    sl = pl.ds(j * 16, 16)
                    msg_v[b, r, sl] = rows_v[b, r, sl] * gate_v[r, sl]
                msg_v[b, r, pl.ds(DOUT, 16)] = ones_col
                return rcarry

            lax.fori_loop(0, DCH, rbody, 0)

            @pl.when(ch + 1 < DNCH)
            def _():
                fire_gate(ch + 1)

            @pl.when(ch + 2 < DNCH)
            def _():
                for cp in idx_copies(ch + 2, b):
                    cp.wait()
                fire_rows(ch + 2, b)
            pltpu.async_copy(msg_v.at[b], acc_sh.at[dstr.at[ch % 4, 0]],
                             sss[b], add=True)
        return carry

    lax.fori_loop(0, DNCH // 2, body, 0)
    for b in range(2):
        pltpu.make_async_copy(
            msg_v.at[b], acc_sh.at[dstr.at[(DNCH - 2 + b) % 4, 0]],
            sss[b]).wait()
    plsc.subcore_barrier()
    pltpu.sync_copy(acc_sh.at[pl.ds(s * 512, 512)],
                    out_h.at[c, pl.ds(s * 512, 512)])


def _stage_e1(acc, tdst, scal, qfv, Wl1, bl1, Wl2p, bl2p,
              Wa1, ba1, Wa2p, ba2p, lv_ref, av_ref):
    a0 = acc[0]
    a1 = acc[1]
    kf = (a0[:, 0:DOUT] / (a0[:, DOUT:DOUT + 1] + 1e-6)
          + a1[:, 0:DOUT] / (a1[:, DOUT:DOUT + 1] + 1e-6)
          + tdst[0])
    kf = kf * _sigmoid(kf)
    h = jnp.concatenate([kf, scal[...], qfv[0]], axis=1)
    ul = jnp.dot(h, Wl1[...], preferred_element_type=F32) + bl1[...]
    ul = ul * _sigmoid(ul)
    lv_ref[0] = jnp.dot(ul, Wl2p[...], preferred_element_type=F32) + bl2p[...]
    ua = jnp.dot(h, Wa1[...], preferred_element_type=F32) + ba1[...]
    ua = ua * _sigmoid(ua)
    av_ref[0] = jnp.dot(ua, Wa2p[...], preferred_element_type=F32) + ba2p[...]


def _stage_e2(lvc, avc, qn, qxT, wrow, ang_ref, lin_ref):
    qw = qn[:, 0:1]
    qx = qn[:, 1:2]
    qy = qn[:, 2:3]
    qz = qn[:, 3:4]

    def irot(vx, vy, vz):
        ux = 2.0 * (qy * vz - qz * vy)
        uy = 2.0 * (qz * vx - qx * vz)
        uz = 2.0 * (qx * vy - qy * vx)
        ox = vx - qw * ux + (qy * uz - qz * uy)
        oy = vy - qw * uy + (qz * ux - qx * uz)
        oz = vz - qw * uz + (qx * uy - qy * ux)
        return ox, oy, oz

    lx, ly, lz = irot(lvc[0], lvc[1], lvc[2])
    ax_, ay_, az_ = irot(avc[0], avc[1], avc[2])
    px = qxT[0:1, :]
    py = qxT[1:2, :]
    pz = qxT[2:3, :]
    obx = py * lz - pz * ly
    oby = pz * lx - px * lz
    obz = px * ly - py * lx
    w = wrow[...]
    lin_ref[:, 0:1] = jnp.sum(w * lx, axis=1, keepdims=True)
    lin_ref[:, 1:2] = jnp.sum(w * ly, axis=1, keepdims=True)
    lin_ref[:, 2:3] = jnp.sum(w * lz, axis=1, keepdims=True)
    ang_ref[:, 0:1] = jnp.sum(w * (ax_ + obx), axis=1, keepdims=True)
    ang_ref[:, 1:2] = jnp.sum(w * (ay_ + oby), axis=1, keepdims=True)
    ang_ref[:, 2:3] = jnp.sum(w * (az_ + obz), axis=1, keepdims=True)


def kernel(Ts, time, query_x, query_f, query_w, key_x_s0, key_f_s0,
           key_x_s1, key_f_s1, edge_src_s0, edge_dst_s0, edge_src_s1,
           edge_dst_s1, Wr0, Wmsg0, Wr1, Wmsg1, Wqt, bqt, Wdst, bdst,
           Wl1, bl1, Wl2, bl2, Wa1, ba1, Wa2, ba2):
    time2 = time.reshape(NT, 1)
    qxT = query_x.T
    vcomp = query_f[:, 32:].reshape(NQ, 8, 3).transpose(2, 1, 0)
    scal = query_f[:, 0:32]
    wrow = query_w.reshape(1, NQ)
    freqs = jnp.asarray(
        np.exp(-np.log(10000.0) * np.arange(TDIM // 2, dtype=np.float32)
               / (TDIM // 2)).reshape(1, TDIM // 2))
    kxc = jnp.concatenate([key_x_s0[:, 0], key_x_s1[:, 0]])
    kyc = jnp.concatenate([key_x_s0[:, 1], key_x_s1[:, 1]])
    kzc = jnp.concatenate([key_x_s0[:, 2], key_x_s1[:, 2]])
    esrc = (jnp.stack([edge_src_s0, edge_src_s1 + NK])
            .reshape(2, NTILE, NCH, CH))
    esrc_d = (jnp.stack([edge_src_s0, edge_src_s1])
              .reshape(2, NTILE, DNCH, 1, DCH))
    edst = jnp.stack([edge_dst_s0, edge_dst_s1]).reshape(2, NTILE, NCH, CH)
    edst_d = (jnp.stack([edge_dst_s0, edge_dst_s1])
              .reshape(2, NTILE, DNCH, 1, DCH))
    Wmsg = jnp.stack([Wmsg0, Wmsg1])
    Wrs = jnp.stack([Wr0, Wr1])
    sig = np.array([1.0 / NB, 4.0 / NB], dtype=np.float32)
    cen = np.stack([np.linspace(0.0, 1.0, NB), np.linspace(0.0, 4.0, NB)])
    cbs = jnp.asarray((cen / sig[:, None]).astype(np.float32)
                      .reshape(2, NB, 1))
    isg = jnp.asarray((1.0 / sig).reshape(2, 1, 1))
    msel = np.zeros((3 * NPRE, 3), dtype=np.float32)
    for i in range(3 * NPRE):
        msel[i, i % 3] = 1.0 / NPRE
    msel = jnp.asarray(msel)
    Wl2p = Wl2[:, 1:] @ msel
    bl2p = (bl2[1:] @ msel).reshape(1, 3)
    Wa2p = Wa2[:, 1:] @ msel
    ba2p = (ba2[1:] @ msel).reshape(1, 3)

    qn, qpos, vrot, tdst = pl.pallas_call(
        _stage_a,
        out_shape=(
            jax.ShapeDtypeStruct((NT, 4), F32),
            jax.ShapeDtypeStruct((3, NT, NQ), F32),
            jax.ShapeDtypeStruct((3, 8, NT, NQ), F32),
            jax.ShapeDtypeStruct((NT, DOUT), F32),
        ),
    )(Ts, time2, qxT, vcomp, Wqt, bqt.reshape(1, TDIM), Wdst,
      bdst.reshape(1, DOUT), freqs)

    kp0, kp1 = pl.pallas_call(
        _stage_ak,
        grid=(NK // 2000,),
        in_specs=[
            pl.BlockSpec((2000, DKEY), lambda i: (i, 0)),
            pl.BlockSpec((2000, DKEY), lambda i: (i, 0)),
            pl.BlockSpec((2, DKEY, DOUT), lambda i: (0, 0, 0)),
        ],
        out_specs=(
            pl.BlockSpec((2000, ACC_W), lambda i: (i, 0)),
            pl.BlockSpec((2000, ACC_W), lambda i: (i, 0)),
        ),
        out_shape=(
            jax.ShapeDtypeStruct((NK, ACC_W), F32),
            jax.ShapeDtypeStruct((NK, ACC_W), F32),
        ),
    )(key_f_s0, key_f_s1, Wmsg)

    qxc = qpos.reshape(3, NR)
    mesh = plsc.VectorSubcoreMesh(core_axis_name="c", subcore_axis_name="s",
                                  num_cores=2, num_subcores=NTILE)
    d2 = pl.kernel(
        _stage_b,
        out_type=jax.ShapeDtypeStruct((2, NTILE, NCH, CH), F32),
        mesh=mesh,
        scratch_types=[
            pltpu.VMEM((NCH, CH), I32),
            pltpu.VMEM((NCH, CH), I32),
            pltpu.VMEM((3 * CH,), F32),
            pltpu.VMEM((3 * CH,), F32),
            pltpu.VMEM((3 * CH,), F32),
            pltpu.VMEM((3 * CH,), F32),
            pltpu.VMEM((NCH, CH), F32),
            pltpu.VMEM_SHARED((2 * NK,), F32),
            pltpu.VMEM_SHARED((2 * NK,), F32),
            pltpu.VMEM_SHARED((2 * NK,), F32),
            pltpu.VMEM_SHARED((NR,), F32),
            pltpu.VMEM_SHARED((NR,), F32),
            pltpu.VMEM_SHARED((NR,), F32),
            pltpu.SemaphoreType.DMA,
            pltpu.SemaphoreType.DMA,
        ],
    )(kxc, kyc, kzc, qxc[0], qxc[1], qxc[2], esrc, edst)

    gate = pl.pallas_call(
        _stage_c,
        grid=(2, NTILE),
        in_specs=[
            pl.BlockSpec((1, 1, 1, NCH * CH), lambda c, i: (c, i, 0, 0)),
            pl.BlockSpec((1, NB, DOUT), lambda c, i: (c, 0, 0)),
            pl.BlockSpec((1, NB, 1), lambda c, i: (c, 0, 0)),
            pl.BlockSpec((1, 1, 1), lambda c, i: (c, 0, 0)),
        ],
        out_specs=pl.BlockSpec((1, NCH * CH, DOUT), lambda c, i: (c, i, 0)),
        out_shape=jax.ShapeDtypeStruct((2, E, DOUT), F32),
    )(d2.reshape(2, NTILE, 1, NCH * CH), Wrs, cbs, isg)

    acc = pl.kernel(
        _stage_d,
        out_type=jax.ShapeDtypeStruct((2, NR, ACC_W), F32),
        mesh=mesh,
        scratch_types=[
            pltpu.VMEM((4, 1, DCH), I32),
            pltpu.VMEM((4, 1, DCH), I32),
            pltpu.VMEM((2, DCH, ACC_W), F32),
            pltpu.VMEM((DCH, DOUT), F32),
            pltpu.VMEM((2, DCH, ACC_W), F32),
            pltpu.VMEM_SHARED((NR, ACC_W), F32),
            pltpu.SemaphoreType.DMA,
            pltpu.SemaphoreType.DMA,
            pltpu.SemaphoreType.DMA,
            pltpu.SemaphoreType.DMA,
            pltpu.SemaphoreType.DMA,
            pltpu.SemaphoreType.DMA,
            pltpu.SemaphoreType.DMA,
            pltpu.SemaphoreType.DMA,
            pltpu.SemaphoreType.DMA,
        ],
    )(kp0, kp1, gate, esrc_d, edst_d)

    qfv = vrot.transpose(2, 3, 1, 0).reshape(NT, NQ, 24)
    lv, av = pl.pallas_call(
        _stage_e1,
        grid=(NT,),
        in_specs=[
            pl.BlockSpec((2, NQ, ACC_W), lambda t: (0, t, 0)),
            pl.BlockSpec((1, 1, DOUT), lambda t: (t, 0, 0)),
            pl.BlockSpec((NQ, 32), lambda t: (0, 0)),
            pl.BlockSpec((1, NQ, 24), lambda t: (t, 0, 0)),
            pl.BlockSpec((DOUT + 56, 128), lambda t: (0, 0)),
            pl.BlockSpec((1, 128), lambda t: (0, 0)),
            pl.BlockSpec((128, 3), lambda t: (0, 0)),
            pl.BlockSpec((1, 3), lambda t: (0, 0)),
            pl.BlockSpec((DOUT + 56, 128), lambda t: (0, 0)),
            pl.BlockSpec((1, 128), lambda t: (0, 0)),
            pl.BlockSpec((128, 3), lambda t: (0, 0)),
            pl.BlockSpec((1, 3), lambda t: (0, 0)),
        ],
        out_specs=(
            pl.BlockSpec((1, NQ, 3), lambda t: (t, 0, 0)),
            pl.BlockSpec((1, NQ, 3), lambda t: (t, 0, 0)),
        ),
        out_shape=(
            jax.ShapeDtypeStruct((NT, NQ, 3), F32),
            jax.ShapeDtypeStruct((NT, NQ, 3), F32),
        ),
    )(acc, tdst.reshape(NT, 1, DOUT), scal, qfv, Wl1, bl1.reshape(1, 128),
      Wl2p, bl2p,
      Wa1, ba1.reshape(1, 128), Wa2p, ba2p)

    lvc = lv.transpose(2, 0, 1)
    avc = av.transpose(2, 0, 1)
    ang_out, lin_out = pl.pallas_call(
        _stage_e2,
        out_shape=(
            jax.ShapeDtypeStruct((NT, 3), F32),
            jax.ShapeDtypeStruct((NT, 3), F32),
        ),
    )(lvc, avc, qn, qxT, wrow)
    return (ang_out, lin_out)


# speedup vs baseline: 1.1875x; 1.1875x over previous
import functools

import numpy as np
import jax
import jax.numpy as jnp
from jax import lax
from jax.experimental import pallas as pl
from jax.experimental.pallas import tpu as pltpu
from jax.experimental.pallas import tpu_sc as plsc

F32 = jnp.float32
I32 = jnp.int32

NT = 16
NQ = 512
NR = NT * NQ
NK = 50000
E = 131072
DKEY = 128
DOUT = 112
TDIM = 128
NB = 16
NPRE = 12
NTILE = 16
CH = 128
NCH = E // (NTILE * CH)
ACC_W = 128


def _sigmoid(x):
    return 1.0 / (1.0 + jnp.exp(-x))


def _stage_a(Ts, time2, qxT, vcomp, Wqt, bqt, Wdst, bdst, freqs,
             qn_ref, qpos_ref, vrot_ref, tdst_ref):
    q = Ts[:, 0:4]
    nrm = jnp.sqrt(jnp.sum(q * q, axis=1, keepdims=True))
    qn = q / (nrm + 1e-8)
    qn_ref[...] = qn
    qw, qx, qy, qz = qn[:, 0:1], qn[:, 1:2], qn[:, 2:3], qn[:, 3:4]

    arg = time2[...] * freqs[...]
    te = jnp.concatenate([jnp.sin(arg), jnp.cos(arg)], axis=1)
    qt = jnp.dot(te, Wqt[...], preferred_element_type=F32) + bqt[...]
    tdst_ref[...] = jnp.dot(qt, Wdst[...], preferred_element_type=F32) + bdst[...]

    def rot(vx, vy, vz):
        tx = 2.0 * (qy * vz - qz * vy)
        ty = 2.0 * (qz * vx - qx * vz)
        tz = 2.0 * (qx * vy - qy * vx)
        ox = vx + qw * tx + (qy * tz - qz * ty)
        oy = vy + qw * ty + (qz * tx - qx * tz)
        oz = vz + qw * tz + (qx * ty - qy * tx)
        return ox, oy, oz

    px, py, pz = rot(qxT[0:1, :], qxT[1:2, :], qxT[2:3, :])
    qpos_ref[0] = px + Ts[:, 4:5]
    qpos_ref[1] = py + Ts[:, 5:6]
    qpos_ref[2] = pz + Ts[:, 6:7]

    for k in range(8):
        ox, oy, oz = rot(vcomp[0, k:k + 1, :], vcomp[1, k:k + 1, :],
                         vcomp[2, k:k + 1, :])
        vrot_ref[0, k] = ox
        vrot_ref[1, k] = oy
        vrot_ref[2, k] = oz


def _stage_ak(kf0, kf1, Wmsg, out0_ref, out1_ref):
    z = jnp.zeros((kf0.shape[0], ACC_W - DOUT), F32)
    p0 = jnp.dot(kf0[...], Wmsg[0], preferred_element_type=F32)
    out0_ref[...] = jnp.concatenate([p0, z], axis=1)
    p1 = jnp.dot(kf1[...], Wmsg[1], preferred_element_type=F32)
    out1_ref[...] = jnp.concatenate([p1, z], axis=1)


def _stage_b(kxc_h, kyc_h, kzc_h, qx0_h, qx1_h, qx2_h, esrc_h, edst_h, out_h,
             src_v, dst_v, kc0, kc1, qc0, qc1, d2_v,
             kx_sh, ky_sh, kz_sh, qx_sh, qy_sh, qz_sh,
             sem0, sem1):
    c = lax.axis_index("c")
    s = lax.axis_index("s")
    pltpu.sync_copy(esrc_h.at[c, s], src_v)
    pltpu.sync_copy(edst_h.at[c, s], dst_v)

    @pl.when(s == 0)
    def _():
        pltpu.sync_copy(kxc_h, kx_sh)
        pltpu.sync_copy(kyc_h, ky_sh)
        pltpu.sync_copy(kzc_h, kz_sh)
        pltpu.sync_copy(qx0_h, qx_sh)
        pltpu.sync_copy(qx1_h, qy_sh)
        pltpu.sync_copy(qx2_h, qz_sh)

    plsc.subcore_barrier()
    sems = (sem0, sem1)
    kcs = (kc0, kc1)
    qcs = (qc0, qc1)
    ktabs = (kx_sh, ky_sh, kz_sh)
    qtabs = (qx_sh, qy_sh, qz_sh)

    def streams(ch, b):
        out = []
        for j in range(3):
            out.append(pltpu.make_async_copy(
                ktabs[j].at[src_v.at[ch]],
                kcs[b].at[pl.ds(j * CH, CH)], sems[b]))
            out.append(pltpu.make_async_copy(
                qtabs[j].at[dst_v.at[ch]],
                qcs[b].at[pl.ds(j * CH, CH)], sems[b]))
        return out

    for b in range(2):
        for st in streams(b, b):
            st.start()

    def body(g, carry):
        for b in range(2):
            ch = 2 * g + b
            for st in streams(ch, b):
                st.wait()
            for grp in range(8):
                o = grp * 16
                kv = kcs[b]
                qv = qcs[b]
                ddx = kv[pl.ds(o, 16)] - qv[pl.ds(o, 16)]
                ddy = kv[pl.ds(CH + o, 16)] - qv[pl.ds(CH + o, 16)]
                ddz = kv[pl.ds(2 * CH + o, 16)] - qv[pl.ds(2 * CH + o, 16)]
                d2_v[ch, pl.ds(o, 16)] = ddx * ddx + ddy * ddy + ddz * ddz
            nxt = ch + 2

            @pl.when(nxt < NCH)
            def _():
                for st in streams(nxt, b):
                    st.start()
        return carry

    lax.fori_loop(0, NCH // 2, body, 0)
    pltpu.sync_copy(d2_v, out_h.at[c, s])


def _stage_c(d2, Wr, cbs, isg, out_ref):
    d = jnp.sqrt(d2[0, 0])
    ds = d * isg[0]
    diff = ds - cbs[0]
    basisT = jnp.exp(-0.5 * diff * diff)
    out_ref[0] = lax.dot_general(
        basisT, Wr[0], dimension_numbers=(((0,), (0,)), ((), ())),
        preferred_element_type=F32)


def _stage_d(kp0_h, kp1_h, gate_h, esrc_h, edst_h, out_h,
             src_v, dst_v, rows_v, gate_v, acc_sh, sr0, sr1, sg):
    c = lax.axis_index("c")
    s = lax.axis_index("s")
    pltpu.sync_copy(esrc_h.at[c, s], src_v)
    pltpu.sync_copy(edst_h.at[c, s], dst_v)
    z16 = jnp.zeros((16,), F32)

    def zbody(r, carry):
        for j in range(8):
            rows_v[0, r, pl.ds(j * 16, 16)] = z16
        return carry

    lax.fori_loop(0, CH, zbody, 0)
    for i in range(4):
        pltpu.sync_copy(rows_v.at[0], acc_sh.at[pl.ds(s * 512 + i * 128, 128)])
    plsc.subcore_barrier()
    ones_col = jnp.where(lax.iota(I32, 16) == 0, 1.0, 0.0).astype(F32)

    srs = (sr0, sr1)
    gtab = gate_h.at[c]

    def fire_rows(ch, b):
        @pl.when(c == 0)
        def _():
            pltpu.async_copy(kp0_h.at[src_v.at[ch]], rows_v.at[b], srs[b])

        @pl.when(c == 1)
        def _():
            pltpu.async_copy(kp1_h.at[src_v.at[ch]], rows_v.at[b], srs[b])

    for b in range(2):
        fire_rows(b, b)
    pltpu.async_copy(gtab.at[pl.ds(s * NCH * CH, CH)], gate_v, sg)

    def body(g, carry):
        for b in range(2):
            ch = 2 * g + b
            pltpu.make_async_copy(kp0_h.at[src_v.at[ch]], rows_v.at[b],
                                  srs[b]).wait()
            pltpu.make_async_copy(gtab.at[pl.ds(s * NCH * CH + ch * CH, CH)],
                                  gate_v, sg).wait()

            def rbody(r, rcarry):
                for j in range(7):
                    sl = pl.ds(j * 16, 16)
                    rows_v[b, r, sl] = rows_v[b, r, sl] * gate_v[r, sl]
                rows_v[b, r, pl.ds(DOUT, 16)] = ones_col
                return rcarry

            lax.fori_loop(0, CH, rbody, 0)

            @pl.when(ch + 1 < NCH)
            def _():
                pltpu.async_copy(
                    gtab.at[pl.ds(s * NCH * CH + (ch + 1) * CH, CH)],
                    gate_v, sg)
            pltpu.sync_copy(rows_v.at[b], acc_sh.at[dst_v.at[ch]], add=True)
            nxt = ch + 2

            @pl.when(nxt < NCH)
            def _():
                fire_rows(nxt, b)
        return carry

    lax.fori_loop(0, NCH // 2, body, 0)
    plsc.subcore_barrier()
    pltpu.sync_copy(acc_sh.at[pl.ds(s * 512, 512)],
                    out_h.at[c, pl.ds(s * 512, 512)])


def _stage_e1(acc, tdst, scal, qfv, Wl1, bl1, Wl2p, bl2p,
              Wa1, ba1, Wa2p, ba2p, lv_ref, av_ref):
    a0 = acc[0]
    a1 = acc[1]
    kf = (a0[:, 0:DOUT] / (a0[:, DOUT:DOUT + 1] + 1e-6)
          + a1[:, 0:DOUT] / (a1[:, DOUT:DOUT + 1] + 1e-6)
          + tdst[0])
    kf = kf * _sigmoid(kf)
    h = jnp.concatenate([kf, scal[...], qfv[0]], axis=1)
    ul = jnp.dot(h, Wl1[...], preferred_element_type=F32) + bl1[...]
    ul = ul * _sigmoid(ul)
    lv_ref[0] = jnp.dot(ul, Wl2p[...], preferred_element_type=F32) + bl2p[...]
    ua = jnp.dot(h, Wa1[...], preferred_element_type=F32) + ba1[...]
    ua = ua * _sigmoid(ua)
    av_ref[0] = jnp.dot(ua, Wa2p[...], preferred_element_type=F32) + ba2p[...]


def _stage_e2(lvc, avc, qn, qxT, wrow, ang_ref, lin_ref):
    qw = qn[:, 0:1]
    qx = qn[:, 1:2]
    qy = qn[:, 2:3]
    qz = qn[:, 3:4]

    def irot(vx, vy, vz):
        ux = 2.0 * (qy * vz - qz * vy)
        uy = 2.0 * (qz * vx - qx * vz)
        uz = 2.0 * (qx * vy - qy * vx)
        ox = vx - qw * ux + (qy * uz - qz * uy)
        oy = vy - qw * uy + (qz * ux - qx * uz)
        oz = vz - qw * uz + (qx * uy - qy * ux)
        return ox, oy, oz

    lx, ly, lz = irot(lvc[0], lvc[1], lvc[2])
    ax_, ay_, az_ = irot(avc[0], avc[1], avc[2])
    px = qxT[0:1, :]
    py = qxT[1:2, :]
    pz = qxT[2:3, :]
    obx = py * lz - pz * ly
    oby = pz * lx - px * lz
    obz = px * ly - py * lx
    w = wrow[...]
    lin_ref[:, 0:1] = jnp.sum(w * lx, axis=1, keepdims=True)
    lin_ref[:, 1:2] = jnp.sum(w * ly, axis=1, keepdims=True)
    lin_ref[:, 2:3] = jnp.sum(w * lz, axis=1, keepdims=True)
    ang_ref[:, 0:1] = jnp.sum(w * (ax_ + obx), axis=1, keepdims=True)
    ang_ref[:, 1:2] = jnp.sum(w * (ay_ + oby), axis=1, keepdims=True)
    ang_ref[:, 2:3] = jnp.sum(w * (az_ + obz), axis=1, keepdims=True)


def kernel(Ts, time, query_x, query_f, query_w, key_x_s0, key_f_s0,
           key_x_s1, key_f_s1, edge_src_s0, edge_dst_s0, edge_src_s1,
           edge_dst_s1, Wr0, Wmsg0, Wr1, Wmsg1, Wqt, bqt, Wdst, bdst,
           Wl1, bl1, Wl2, bl2, Wa1, ba1, Wa2, ba2):
    time2 = time.reshape(NT, 1)
    qxT = query_x.T
    vcomp = query_f[:, 32:].reshape(NQ, 8, 3).transpose(2, 1, 0)
    scal = query_f[:, 0:32]
    wrow = query_w.reshape(1, NQ)
    freqs = jnp.asarray(
        np.exp(-np.log(10000.0) * np.arange(TDIM // 2, dtype=np.float32)
               / (TDIM // 2)).reshape(1, TDIM // 2))
    kxc = jnp.concatenate([key_x_s0[:, 0], key_x_s1[:, 0]])
    kyc = jnp.concatenate([key_x_s0[:, 1], key_x_s1[:, 1]])
    kzc = jnp.concatenate([key_x_s0[:, 2], key_x_s1[:, 2]])
    esrc = (jnp.stack([edge_src_s0, edge_src_s1 + NK])
            .reshape(2, NTILE, NCH, CH))
    esrc_d = jnp.stack([edge_src_s0, edge_src_s1]).reshape(2, NTILE, NCH, CH)
    edst = jnp.stack([edge_dst_s0, edge_dst_s1]).reshape(2, NTILE, NCH, CH)
    Wmsg = jnp.stack([Wmsg0, Wmsg1])
    Wrs = jnp.stack([Wr0, Wr1])
    sig = np.array([1.0 / NB, 4.0 / NB], dtype=np.float32)
    cen = np.stack([np.linspace(0.0, 1.0, NB), np.linspace(0.0, 4.0, NB)])
    cbs = jnp.asarray((cen / sig[:, None]).astype(np.float32)
                      .reshape(2, NB, 1))
    isg = jnp.asarray((1.0 / sig).reshape(2, 1, 1))
    msel = np.zeros((3 * NPRE, 3), dtype=np.float32)
    for i in range(3 * NPRE):
        msel[i, i % 3] = 1.0 / NPRE
    msel = jnp.asarray(msel)
    Wl2p = Wl2[:, 1:] @ msel
    bl2p = (bl2[1:] @ msel).reshape(1, 3)
    Wa2p = Wa2[:, 1:] @ msel
    ba2p = (ba2[1:] @ msel).reshape(1, 3)

    qn, qpos, vrot, tdst = pl.pallas_call(
        _stage_a,
        out_shape=(
            jax.ShapeDtypeStruct((NT, 4), F32),
            jax.ShapeDtypeStruct((3, NT, NQ), F32),
            jax.ShapeDtypeStruct((3, 8, NT, NQ), F32),
            jax.ShapeDtypeStruct((NT, DOUT), F32),
        ),
    )(Ts, time2, qxT, vcomp, Wqt, bqt.reshape(1, TDIM), Wdst,
      bdst.reshape(1, DOUT), freqs)

    kp0, kp1 = pl.pallas_call(
        _stage_ak,
        grid=(NK // 2000,),
        in_specs=[
            pl.BlockSpec((2000, DKEY), lambda i: (i, 0)),
            pl.BlockSpec((2000, DKEY), lambda i: (i, 0)),
            pl.BlockSpec((2, DKEY, DOUT), lambda i: (0, 0, 0)),
        ],
        out_specs=(
            pl.BlockSpec((2000, ACC_W), lambda i: (i, 0)),
            pl.BlockSpec((2000, ACC_W), lambda i: (i, 0)),
        ),
        out_shape=(
            jax.ShapeDtypeStruct((NK, ACC_W), F32),
            jax.ShapeDtypeStruct((NK, ACC_W), F32),
        ),
    )(key_f_s0, key_f_s1, Wmsg)

    qxc = qpos.reshape(3, NR)
    mesh = plsc.VectorSubcoreMesh(core_axis_name="c", subcore_axis_name="s",
                                  num_cores=2, num_subcores=NTILE)
    d2 = pl.kernel(
        _stage_b,
        out_type=jax.ShapeDtypeStruct((2, NTILE, NCH, CH), F32),
        mesh=mesh,
        scratch_types=[
            pltpu.VMEM((NCH, CH), I32),
            pltpu.VMEM((NCH, CH), I32),
            pltpu.VMEM((3 * CH,), F32),
            pltpu.VMEM((3 * CH,), F32),
            pltpu.VMEM((3 * CH,), F32),
            pltpu.VMEM((3 * CH,), F32),
            pltpu.VMEM((NCH, CH), F32),
            pltpu.VMEM_SHARED((2 * NK,), F32),
            pltpu.VMEM_SHARED((2 * NK,), F32),
            pltpu.VMEM_SHARED((2 * NK,), F32),
            pltpu.VMEM_SHARED((NR,), F32),
            pltpu.VMEM_SHARED((NR,), F32),
            pltpu.VMEM_SHARED((NR,), F32),
            pltpu.SemaphoreType.DMA,
            pltpu.SemaphoreType.DMA,
        ],
    )(kxc, kyc, kzc, qxc[0], qxc[1], qxc[2], esrc, edst)

    gate = pl.pallas_call(
        _stage_c,
        grid=(2, NTILE),
        in_specs=[
            pl.BlockSpec((1, 1, 1, NCH * CH), lambda c, i: (c, i, 0, 0)),
            pl.BlockSpec((1, NB, DOUT), lambda c, i: (c, 0, 0)),
            pl.BlockSpec((1, NB, 1), lambda c, i: (c, 0, 0)),
            pl.BlockSpec((1, 1, 1), lambda c, i: (c, 0, 0)),
        ],
        out_specs=pl.BlockSpec((1, NCH * CH, DOUT), lambda c, i: (c, i, 0)),
        out_shape=jax.ShapeDtypeStruct((2, E, DOUT), F32),
    )(d2.reshape(2, NTILE, 1, NCH * CH), Wrs, cbs, isg)

    acc = pl.kernel(
        _stage_d,
        out_type=jax.ShapeDtypeStruct((2, NR, ACC_W), F32),
        mesh=mesh,
        scratch_types=[
            pltpu.VMEM((NCH, CH), I32),
            pltpu.VMEM((NCH, CH), I32),
            pltpu.VMEM((2, CH, ACC_W), F32),
            pltpu.VMEM((CH, DOUT), F32),
            pltpu.VMEM_SHARED((NR, ACC_W), F32),
            pltpu.SemaphoreType.DMA,
            pltpu.SemaphoreType.DMA,
            pltpu.SemaphoreType.DMA,
        ],
    )(kp0, kp1, gate, esrc_d, edst)

    qfv = vrot.transpose(2, 3, 1, 0).reshape(NT, NQ, 24)
    lv, av = pl.pallas_call(
        _stage_e1,
        grid=(NT,),
        in_specs=[
            pl.BlockSpec((2, NQ, ACC_W), lambda t: (0, t, 0)),
            pl.BlockSpec((1, 1, DOUT), lambda t: (t, 0, 0)),
            pl.BlockSpec((NQ, 32), lambda t: (0, 0)),
            pl.BlockSpec((1, NQ, 24), lambda t: (t, 0, 0)),
            pl.BlockSpec((DOUT + 56, 128), lambda t: (0, 0)),
            pl.BlockSpec((1, 128), lambda t: (0, 0)),
            pl.BlockSpec((128, 3), lambda t: (0, 0)),
            pl.BlockSpec((1, 3), lambda t: (0, 0)),
            pl.BlockSpec((DOUT + 56, 128), lambda t: (0, 0)),
            pl.BlockSpec((1, 128), lambda t: (0, 0)),
            pl.BlockSpec((128, 3), lambda t: (0, 0)),
            pl.BlockSpec((1, 3), lambda t: (0, 0)),
        ],
        out_specs=(
            pl.BlockSpec((1, NQ, 3), lambda t: (t, 0, 0)),
            pl.BlockSpec((1, NQ, 3), lambda t: (t, 0, 0)),
        ),
        out_shape=(
            jax.ShapeDtypeStruct((NT, NQ, 3), F32),
            jax.ShapeDtypeStruct((NT, NQ, 3), F32),
        ),
    )(acc, tdst.reshape(NT, 1, DOUT), scal, qfv, Wl1, bl1.reshape(1, 128),
      Wl2p, bl2p,
      Wa1, ba1.reshape(1, 128), Wa2p, ba2p)

    lvc = lv.transpose(2, 0, 1)
    avc = av.transpose(2, 0, 1)
    ang_out, lin_out = pl.pallas_call(
        _stage_e2,
        out_shape=(
            jax.ShapeDtypeStruct((NT, 3), F32),
            jax.ShapeDtypeStruct((NT, 3), F32),
        ),
    )(lvc, avc, qn, qxT, wrow)
    return (ang_out, lin_out)
